# baseline (speedup 1.0000x reference)
import jax
import jax.numpy as jnp
from jax.experimental import pallas as pl

_FRAMES = 12
_AFTER = 6
_DUR = 2
_K = 49
_ITERS = 10
_N = 98


def _seg_body(dm_ref, first_ref, cls_ref, x_ref, out_ref):
    N, K = _N, _K
    Dm = dm_ref[0]
    first = first_ref[0, 0, 0]
    X = x_ref[0]

    cls2 = cls_ref[0]
    cls_emb = jnp.mean(cls2, axis=0, keepdims=True)

    iota_ncol = jax.lax.broadcasted_iota(jnp.int32, (N, 1), 0)
    iota_nrow = jax.lax.broadcasted_iota(jnp.int32, (1, N), 1)
    iota_kcol = jax.lax.broadcasted_iota(jnp.int32, (K, 1), 0)
    iota_krow = jax.lax.broadcasted_iota(jnp.int32, (1, K), 1)

    ids = jnp.where(iota_krow == 0, first, 0)
    mind = jnp.sum(jnp.where(iota_ncol == first, Dm, 0.0),
                   axis=0, keepdims=True)

    def init_body(i, carry):
        ids, mind = carry
        m = jnp.max(mind)
        nxt = jnp.min(jnp.where(mind == m, iota_nrow, N))
        ids = jnp.where(iota_krow == i, nxt, ids)
        row = jnp.sum(jnp.where(iota_ncol == nxt, Dm, 0.0),
                      axis=0, keepdims=True)
        return ids, jnp.minimum(mind, row)

    ids, _ = jax.lax.fori_loop(1, K, init_body, (ids, mind))

    BIG = jnp.int32(1 << 30)
    INF = jnp.float32(jnp.inf)

    def iter_body(_, ids):
        P = (jnp.transpose(ids) == iota_nrow)
        ordj = jnp.min(jnp.where(P, iota_kcol, BIG),
                       axis=0, keepdims=True)
        in_set = ordj < BIG
        masked = jnp.where(in_set, Dm, INF)
        minv = jnp.min(masked, axis=1, keepdims=True)
        assign = jnp.min(jnp.where(masked == minv, ordj, BIG),
                         axis=1, keepdims=True)
        member = assign == iota_krow
        cost = jax.lax.dot_general(
            Dm, member.astype(jnp.float32), (((1,), (0,)), ((), ())),
            preferred_element_type=jnp.float32)
        cost = jnp.where(member, cost, 1e10)
        minc = jnp.min(cost, axis=0, keepdims=True)
        newids = jnp.min(jnp.where(cost == minc, iota_ncol, N),
                         axis=0, keepdims=True)
        return newids

    ids = jax.lax.fori_loop(0, _ITERS, iter_body, ids)

    ids_col = jnp.transpose(ids)
    lt = (ids < ids_col).astype(jnp.int32)
    eq_jlt = ((ids == ids_col) & (iota_krow < iota_kcol)).astype(jnp.int32)
    rank = jnp.sum(lt + eq_jlt, axis=1, keepdims=True)
    srt = jnp.sum(jnp.where(rank == iota_krow, ids_col, 0),
                  axis=0, keepdims=True)

    Ps = (jnp.transpose(srt) == iota_nrow).astype(jnp.float32)
    centers = jax.lax.dot_general(
        Ps, X, (((1,), (0,)), ((), ())),
        precision=jax.lax.Precision.HIGHEST,
        preferred_element_type=jnp.float32)

    out_ref[0, 0:1, :] = cls_emb
    out_ref[0, 1:, :] = centers


def _dists(X):
    sq = jnp.sum(X * X, axis=-1)
    d2 = sq[:, None] + sq[None, :] - 2.0 * (X @ X.T)
    Dm = jnp.sqrt(jnp.maximum(d2, 0.0) + 1e-12)
    first = jnp.argmin(jnp.sum(Dm, axis=1)).astype(jnp.int32)
    return Dm, first


def kernel(x):
    L, Bt, D = x.shape
    B = Bt // _FRAMES
    S = B * _AFTER

    xp = jnp.transpose(x, (1, 0, 2))
    cls = xp[:, 0, :].reshape(B, _FRAMES, 1, D)
    res = xp[:, 1:, :].reshape(B, _FRAMES, L - 1, D)
    segs = jnp.split(res, _AFTER, axis=1)
    res_tmp = jnp.concatenate(segs, axis=0).reshape(S, _N, D)
    res_tmp = jax.lax.stop_gradient(res_tmp)
    cls_segs = jnp.split(cls, _AFTER, axis=1)
    cls_cat = jnp.concatenate(cls_segs, axis=0).reshape(S, _DUR, D)

    Dm, first = jax.vmap(_dists)(res_tmp)

    out = pl.pallas_call(
        _seg_body,
        grid=(S,),
        in_specs=[
            pl.BlockSpec((1, _N, _N), lambda i: (i, 0, 0)),
            pl.BlockSpec((1, 1, 1), lambda i: (i, 0, 0)),
            pl.BlockSpec((1, _DUR, D), lambda i: (i, 0, 0)),
            pl.BlockSpec((1, _N, D), lambda i: (i, 0, 0)),
        ],
        out_specs=pl.BlockSpec((1, L, D), lambda i: (i, 0, 0)),
        out_shape=jax.ShapeDtypeStruct((S, L, D), jnp.float32),
    )(Dm, first.reshape(S, 1, 1), cls_cat, res_tmp)
    return jnp.transpose(out, (1, 0, 2))


# speedup vs baseline: 4.5240x; 4.5240x over previous
import jax
import jax.numpy as jnp
from jax.experimental import pallas as pl

_FRAMES = 12
_AFTER = 6
_DUR = 2
_K = 49
_ITERS = 10
_N = 98
_SB = 8


def _seg_body(dm_ref, first_ref, cls_ref, x_ref, out_ref):
    N, K = _N, _K
    Dm = dm_ref[...]
    first = first_ref[...]
    X = x_ref[...]

    cls_emb = jnp.mean(cls_ref[...], axis=1, keepdims=True)

    iota_n1 = jax.lax.broadcasted_iota(jnp.int32, (1, N, 1), 1)
    iota_n2 = jax.lax.broadcasted_iota(jnp.int32, (1, 1, N), 2)
    iota_k1 = jax.lax.broadcasted_iota(jnp.int32, (1, K, 1), 1)
    iota_k2 = jax.lax.broadcasted_iota(jnp.int32, (1, 1, K), 2)

    ids = jnp.where(iota_k2 == 0, first, 0)
    mind = jnp.sum(jnp.where(iota_n1 == first, Dm, 0.0),
                   axis=1, keepdims=True)

    def init_body(i, carry):
        ids, mind = carry
        m = jnp.max(mind, axis=2, keepdims=True)
        nxt = jnp.min(jnp.where(mind == m, iota_n2, N),
                      axis=2, keepdims=True)
        ids = jnp.where(iota_k2 == i, nxt, ids)
        row = jnp.sum(jnp.where(iota_n1 == nxt, Dm, 0.0),
                      axis=1, keepdims=True)
        return ids, jnp.minimum(mind, row)

    ids, _ = jax.lax.fori_loop(1, K, init_body, (ids, mind))

    BIG = jnp.int32(1 << 30)
    INF = jnp.float32(jnp.inf)

    def iter_body(_, ids):
        P = jnp.transpose(ids, (0, 2, 1)) == iota_n2
        ordj = jnp.min(jnp.where(P, iota_k1, BIG),
                       axis=1, keepdims=True)
        in_set = ordj < BIG
        masked = jnp.where(in_set, Dm, INF)
        minv = jnp.min(masked, axis=2, keepdims=True)
        assign = jnp.min(jnp.where(masked == minv, ordj, BIG),
                         axis=2, keepdims=True)
        member = assign == iota_k2
        cost = jax.lax.dot_general(
            Dm, member.astype(jnp.float32), (((2,), (1,)), ((0,), (0,))),
            preferred_element_type=jnp.float32)
        cost = jnp.where(member, cost, 1e10)
        minc = jnp.min(cost, axis=1, keepdims=True)
        newids = jnp.min(jnp.where(cost == minc, iota_n1, N),
                         axis=1, keepdims=True)
        return newids

    ids = jax.lax.fori_loop(0, _ITERS, iter_body, ids)

    ids_col = jnp.transpose(ids, (0, 2, 1))
    lt = (ids < ids_col).astype(jnp.int32)
    eq_jlt = ((ids == ids_col) & (iota_k2 < iota_k1)).astype(jnp.int32)
    rank = jnp.sum(lt + eq_jlt, axis=2, keepdims=True)
    srt = jnp.sum(jnp.where(rank == iota_k2, ids_col, 0),
                  axis=1, keepdims=True)

    Ps = (jnp.transpose(srt, (0, 2, 1)) == iota_n2).astype(jnp.float32)
    centers = jax.lax.dot_general(
        Ps, X, (((2,), (1,)), ((0,), (0,))),
        precision=jax.lax.Precision.HIGHEST,
        preferred_element_type=jnp.float32)

    out_ref[:, 0:1, :] = cls_emb
    out_ref[:, 1:, :] = centers


def _dists(X):
    sq = jnp.sum(X * X, axis=-1)
    d2 = sq[:, None] + sq[None, :] - 2.0 * (X @ X.T)
    Dm = jnp.sqrt(jnp.maximum(d2, 0.0) + 1e-12)
    first = jnp.argmin(jnp.sum(Dm, axis=1)).astype(jnp.int32)
    return Dm, first


def kernel(x):
    L, Bt, D = x.shape
    B = Bt // _FRAMES
    S = B * _AFTER

    xp = jnp.transpose(x, (1, 0, 2))
    cls = xp[:, 0, :].reshape(B, _FRAMES, 1, D)
    res = xp[:, 1:, :].reshape(B, _FRAMES, L - 1, D)
    segs = jnp.split(res, _AFTER, axis=1)
    res_tmp = jnp.concatenate(segs, axis=0).reshape(S, _N, D)
    res_tmp = jax.lax.stop_gradient(res_tmp)
    cls_segs = jnp.split(cls, _AFTER, axis=1)
    cls_cat = jnp.concatenate(cls_segs, axis=0).reshape(S, _DUR, D)

    Dm, first = jax.vmap(_dists)(res_tmp)

    out = pl.pallas_call(
        _seg_body,
        grid=(S // _SB,),
        in_specs=[
            pl.BlockSpec((_SB, _N, _N), lambda i: (i, 0, 0)),
            pl.BlockSpec((_SB, 1, 1), lambda i: (i, 0, 0)),
            pl.BlockSpec((_SB, _DUR, D), lambda i: (i, 0, 0)),
            pl.BlockSpec((_SB, _N, D), lambda i: (i, 0, 0)),
        ],
        out_specs=pl.BlockSpec((_SB, L, D), lambda i: (i, 0, 0)),
        out_shape=jax.ShapeDtypeStruct((S, L, D), jnp.float32),
    )(Dm, first.reshape(S, 1, 1), cls_cat, res_tmp)
    return jnp.transpose(out, (1, 0, 2))


# speedup vs baseline: 5.6253x; 1.2434x over previous
import jax
import jax.numpy as jnp
from jax.experimental import pallas as pl

_FRAMES = 12
_AFTER = 6
_DUR = 2
_K = 49
_ITERS = 10
_N = 98
_SB = 16


def _seg_body(dm_ref, first_ref, cls_ref, x_ref, out_ref):
    N, K = _N, _K
    Dm = dm_ref[...]
    first = first_ref[...]
    X = x_ref[...]

    cls_emb = jnp.mean(cls_ref[...], axis=1, keepdims=True)

    iota_n1 = jax.lax.broadcasted_iota(jnp.int32, (1, N, 1), 1)
    iota_n2 = jax.lax.broadcasted_iota(jnp.int32, (1, 1, N), 2)
    iota_k1 = jax.lax.broadcasted_iota(jnp.int32, (1, K, 1), 1)
    iota_k2 = jax.lax.broadcasted_iota(jnp.int32, (1, 1, K), 2)

    ids = jnp.where(iota_k2 == 0, first, 0)
    mind = jnp.sum(jnp.where(iota_n1 == first, Dm, 0.0),
                   axis=1, keepdims=True)

    def init_body(i, carry):
        ids, mind = carry
        m = jnp.max(mind, axis=2, keepdims=True)
        nxt = jnp.min(jnp.where(mind == m, iota_n2, N),
                      axis=2, keepdims=True)
        ids = jnp.where(iota_k2 == i, nxt, ids)
        row = jnp.sum(jnp.where(iota_n1 == nxt, Dm, 0.0),
                      axis=1, keepdims=True)
        return ids, jnp.minimum(mind, row)

    ids, _ = jax.lax.fori_loop(1, K, init_body, (ids, mind))

    BIG = jnp.int32(1 << 30)
    INF = jnp.float32(jnp.inf)

    def iter_body(_, ids):
        P = jnp.transpose(ids, (0, 2, 1)) == iota_n2
        ordj = jnp.min(jnp.where(P, iota_k1, BIG),
                       axis=1, keepdims=True)
        in_set = ordj < BIG
        masked = jnp.where(in_set, Dm, INF)
        minv = jnp.min(masked, axis=2, keepdims=True)
        assign = jnp.min(jnp.where(masked == minv, ordj, BIG),
                         axis=2, keepdims=True)
        member = assign == iota_k2
        cost = jax.lax.dot_general(
            Dm, member.astype(jnp.float32), (((2,), (1,)), ((0,), (0,))),
            preferred_element_type=jnp.float32)
        cost = jnp.where(member, cost, 1e10)
        minc = jnp.min(cost, axis=1, keepdims=True)
        newids = jnp.min(jnp.where(cost == minc, iota_n1, N),
                         axis=1, keepdims=True)
        return newids

    ids = jax.lax.fori_loop(0, _ITERS, iter_body, ids)

    ids_col = jnp.transpose(ids, (0, 2, 1))
    lt = (ids < ids_col).astype(jnp.int32)
    eq_jlt = ((ids == ids_col) & (iota_k2 < iota_k1)).astype(jnp.int32)
    rank = jnp.sum(lt + eq_jlt, axis=2, keepdims=True)
    srt = jnp.sum(jnp.where(rank == iota_k2, ids_col, 0),
                  axis=1, keepdims=True)

    Ps = (jnp.transpose(srt, (0, 2, 1)) == iota_n2).astype(jnp.float32)
    centers = jax.lax.dot_general(
        Ps, X, (((2,), (1,)), ((0,), (0,))),
        precision=jax.lax.Precision.HIGHEST,
        preferred_element_type=jnp.float32)

    out_ref[:, 0:1, :] = cls_emb
    out_ref[:, 1:, :] = centers


def _dists(X):
    sq = jnp.sum(X * X, axis=-1)
    d2 = sq[:, None] + sq[None, :] - 2.0 * (X @ X.T)
    Dm = jnp.sqrt(jnp.maximum(d2, 0.0) + 1e-12)
    first = jnp.argmin(jnp.sum(Dm, axis=1)).astype(jnp.int32)
    return Dm, first


def kernel(x):
    L, Bt, D = x.shape
    B = Bt // _FRAMES
    S = B * _AFTER

    xp = jnp.transpose(x, (1, 0, 2))
    cls = xp[:, 0, :].reshape(B, _FRAMES, 1, D)
    res = xp[:, 1:, :].reshape(B, _FRAMES, L - 1, D)
    segs = jnp.split(res, _AFTER, axis=1)
    res_tmp = jnp.concatenate(segs, axis=0).reshape(S, _N, D)
    res_tmp = jax.lax.stop_gradient(res_tmp)
    cls_segs = jnp.split(cls, _AFTER, axis=1)
    cls_cat = jnp.concatenate(cls_segs, axis=0).reshape(S, _DUR, D)

    Dm, first = jax.vmap(_dists)(res_tmp)

    out = pl.pallas_call(
        _seg_body,
        grid=(S // _SB,),
        in_specs=[
            pl.BlockSpec((_SB, _N, _N), lambda i: (i, 0, 0)),
            pl.BlockSpec((_SB, 1, 1), lambda i: (i, 0, 0)),
            pl.BlockSpec((_SB, _DUR, D), lambda i: (i, 0, 0)),
            pl.BlockSpec((_SB, _N, D), lambda i: (i, 0, 0)),
        ],
        out_specs=pl.BlockSpec((_SB, L, D), lambda i: (i, 0, 0)),
        out_shape=jax.ShapeDtypeStruct((S, L, D), jnp.float32),
    )(Dm, first.reshape(S, 1, 1), cls_cat, res_tmp)
    return jnp.transpose(out, (1, 0, 2))


# speedup vs baseline: 6.2668x; 1.1140x over previous
import jax
import jax.numpy as jnp
from jax.experimental import pallas as pl

_FRAMES = 12
_AFTER = 6
_DUR = 2
_K = 49
_ITERS = 10
_N = 98
_SB = 32


def _seg_body(dm_ref, first_ref, cls_ref, x_ref, out_ref):
    N, K = _N, _K
    Dm = dm_ref[...]
    first = first_ref[...]
    X = x_ref[...]

    cls_emb = jnp.mean(cls_ref[...], axis=1, keepdims=True)

    iota_n1 = jax.lax.broadcasted_iota(jnp.int32, (1, N, 1), 1)
    iota_n2 = jax.lax.broadcasted_iota(jnp.int32, (1, 1, N), 2)
    iota_k1 = jax.lax.broadcasted_iota(jnp.int32, (1, K, 1), 1)
    iota_k2 = jax.lax.broadcasted_iota(jnp.int32, (1, 1, K), 2)

    ids = jnp.where(iota_k2 == 0, first, 0)
    mind = jnp.sum(jnp.where(iota_n1 == first, Dm, 0.0),
                   axis=1, keepdims=True)

    def init_body(i, carry):
        ids, mind = carry
        m = jnp.max(mind, axis=2, keepdims=True)
        nxt = jnp.min(jnp.where(mind == m, iota_n2, N),
                      axis=2, keepdims=True)
        ids = jnp.where(iota_k2 == i, nxt, ids)
        row = jnp.sum(jnp.where(iota_n1 == nxt, Dm, 0.0),
                      axis=1, keepdims=True)
        return ids, jnp.minimum(mind, row)

    ids, _ = jax.lax.fori_loop(1, K, init_body, (ids, mind))

    BIG = jnp.int32(1 << 30)
    INF = jnp.float32(jnp.inf)

    def iter_body(_, ids):
        P = jnp.transpose(ids, (0, 2, 1)) == iota_n2
        ordj = jnp.min(jnp.where(P, iota_k1, BIG),
                       axis=1, keepdims=True)
        in_set = ordj < BIG
        masked = jnp.where(in_set, Dm, INF)
        minv = jnp.min(masked, axis=2, keepdims=True)
        assign = jnp.min(jnp.where(masked == minv, ordj, BIG),
                         axis=2, keepdims=True)
        member = assign == iota_k2
        cost = jax.lax.dot_general(
            Dm, member.astype(jnp.float32), (((2,), (1,)), ((0,), (0,))),
            preferred_element_type=jnp.float32)
        cost = jnp.where(member, cost, 1e10)
        minc = jnp.min(cost, axis=1, keepdims=True)
        newids = jnp.min(jnp.where(cost == minc, iota_n1, N),
                         axis=1, keepdims=True)
        return newids

    ids = jax.lax.fori_loop(0, _ITERS, iter_body, ids)

    ids_col = jnp.transpose(ids, (0, 2, 1))
    lt = (ids < ids_col).astype(jnp.int32)
    eq_jlt = ((ids == ids_col) & (iota_k2 < iota_k1)).astype(jnp.int32)
    rank = jnp.sum(lt + eq_jlt, axis=2, keepdims=True)
    srt = jnp.sum(jnp.where(rank == iota_k2, ids_col, 0),
                  axis=1, keepdims=True)

    Ps = (jnp.transpose(srt, (0, 2, 1)) == iota_n2).astype(jnp.float32)
    centers = jax.lax.dot_general(
        Ps, X, (((2,), (1,)), ((0,), (0,))),
        precision=jax.lax.Precision.HIGHEST,
        preferred_element_type=jnp.float32)

    out_ref[:, 0:1, :] = cls_emb
    out_ref[:, 1:, :] = centers


def _dists(X):
    sq = jnp.sum(X * X, axis=-1)
    d2 = sq[:, None] + sq[None, :] - 2.0 * (X @ X.T)
    Dm = jnp.sqrt(jnp.maximum(d2, 0.0) + 1e-12)
    first = jnp.argmin(jnp.sum(Dm, axis=1)).astype(jnp.int32)
    return Dm, first


def kernel(x):
    L, Bt, D = x.shape
    B = Bt // _FRAMES
    S = B * _AFTER

    xp = jnp.transpose(x, (1, 0, 2))
    cls = xp[:, 0, :].reshape(B, _FRAMES, 1, D)
    res = xp[:, 1:, :].reshape(B, _FRAMES, L - 1, D)
    segs = jnp.split(res, _AFTER, axis=1)
    res_tmp = jnp.concatenate(segs, axis=0).reshape(S, _N, D)
    res_tmp = jax.lax.stop_gradient(res_tmp)
    cls_segs = jnp.split(cls, _AFTER, axis=1)
    cls_cat = jnp.concatenate(cls_segs, axis=0).reshape(S, _DUR, D)

    Dm, first = jax.vmap(_dists)(res_tmp)

    out = pl.pallas_call(
        _seg_body,
        grid=(S // _SB,),
        in_specs=[
            pl.BlockSpec((_SB, _N, _N), lambda i: (i, 0, 0)),
            pl.BlockSpec((_SB, 1, 1), lambda i: (i, 0, 0)),
            pl.BlockSpec((_SB, _DUR, D), lambda i: (i, 0, 0)),
            pl.BlockSpec((_SB, _N, D), lambda i: (i, 0, 0)),
        ],
        out_specs=pl.BlockSpec((_SB, L, D), lambda i: (i, 0, 0)),
        out_shape=jax.ShapeDtypeStruct((S, L, D), jnp.float32),
    )(Dm, first.reshape(S, 1, 1), cls_cat, res_tmp)
    return jnp.transpose(out, (1, 0, 2))
